# baseline (speedup 1.0000x reference)
import functools

import jax
import jax.numpy as jnp
from jax import lax
from jax.experimental import pallas as pl
from jax.experimental.pallas import tpu as pltpu
import jax.experimental.pallas.tpu_sc as plsc

N = 10000
E = 160000
D_IN = 256
NEG = 0.2

LANES = 128
ROWS = 1280
EP = ROWS * LANES
NTILES = 32
RPT = ROWS // NTILES
RPH = ROWS // 2 // 16
NP = 10240
STR = NP // 16
BN = 200
GRID = N // BN


def _tc_a_body(x_ref, w_ref, asr_ref, adr_ref, xw_ref, ab_ref, m_ref):
    i = pl.program_id(0)
    xw = jnp.dot(x_ref[...], w_ref[...], preferred_element_type=jnp.float32)
    xw_ref[...] = xw
    s = (xw * asr_ref[...]).reshape(BN, 4, 128).sum(-1)
    d = (xw * adr_ref[...]).reshape(BN, 4, 128).sum(-1)
    ab = jnp.concatenate([s, d], axis=1)
    ab_ref[...] = ab
    cur = jnp.max(ab, axis=0, keepdims=True)

    @pl.when(i == 0)
    def _():
        m_ref[...] = cur

    @pl.when(i > 0)
    def _():
        m_ref[...] = jnp.maximum(m_ref[...], cur)


def _tc_a(x, W1, a1s, a1d):
    return pl.pallas_call(
        _tc_a_body,
        grid=(GRID,),
        in_specs=[
            pl.BlockSpec((BN, D_IN), lambda i: (i, 0)),
            pl.BlockSpec((D_IN, 512), lambda i: (0, 0)),
            pl.BlockSpec((1, 512), lambda i: (0, 0)),
            pl.BlockSpec((1, 512), lambda i: (0, 0)),
        ],
        out_specs=[
            pl.BlockSpec((BN, 512), lambda i: (i, 0)),
            pl.BlockSpec((BN, 8), lambda i: (i, 0)),
            pl.BlockSpec((1, 8), lambda i: (0, 0)),
        ],
        out_shape=[
            jax.ShapeDtypeStruct((N, 512), jnp.float32),
            jax.ShapeDtypeStruct((N, 8), jnp.float32),
            jax.ShapeDtypeStruct((1, 8), jnp.float32),
        ],
    )(x, W1, a1s, a1d)


def _tc_b_body(acc_ref, den_ref, b1_ref, w2_ref, a2s_ref, a2d_ref,
               xw_ref, ab_ref, m_ref):
    i = pl.program_id(0)
    den = den_ref[0] + den_ref[1]
    xw = jnp.zeros((BN, 512), jnp.float32)
    for p in range(4):
        hp = jnp.concatenate(
            [acc_ref[0, 2 * p] + acc_ref[1, 2 * p],
             acc_ref[0, 2 * p + 1] + acc_ref[1, 2 * p + 1]], axis=1)
        hp = hp / (den[:, p:p + 1] + 1e-16) + b1_ref[0, p * 128:(p + 1) * 128]
        hp = jnp.maximum(hp, 0.0)
        xw = xw + jnp.dot(hp, w2_ref[p], preferred_element_type=jnp.float32)
    xw_ref[...] = xw
    s = (xw * a2s_ref[...]).sum(-1, keepdims=True)
    d = (xw * a2d_ref[...]).sum(-1, keepdims=True)
    z = jnp.zeros((BN, 3), jnp.float32)
    ab = jnp.concatenate([s, z, d, z], axis=1)
    ab_ref[...] = ab
    cur = jnp.max(ab, axis=0, keepdims=True)

    @pl.when(i == 0)
    def _():
        m_ref[...] = cur

    @pl.when(i > 0)
    def _():
        m_ref[...] = jnp.maximum(m_ref[...], cur)


def _tc_b(acc1, den1, b1, W2r, a2s, a2d):
    return pl.pallas_call(
        _tc_b_body,
        grid=(GRID,),
        in_specs=[
            pl.BlockSpec((2, 8, BN, 64), lambda i: (0, 0, i, 0)),
            pl.BlockSpec((2, BN, 16), lambda i: (0, i, 0)),
            pl.BlockSpec((1, 512), lambda i: (0, 0)),
            pl.BlockSpec((4, 128, 512), lambda i: (0, 0, 0)),
            pl.BlockSpec((1, 512), lambda i: (0, 0)),
            pl.BlockSpec((1, 512), lambda i: (0, 0)),
        ],
        out_specs=[
            pl.BlockSpec((BN, 512), lambda i: (i, 0)),
            pl.BlockSpec((BN, 8), lambda i: (i, 0)),
            pl.BlockSpec((1, 8), lambda i: (0, 0)),
        ],
        out_shape=[
            jax.ShapeDtypeStruct((N, 512), jnp.float32),
            jax.ShapeDtypeStruct((N, 8), jnp.float32),
            jax.ShapeDtypeStruct((1, 8), jnp.float32),
        ],
    )(acc1, den1, b1, W2r, a2s, a2d)


def _tc_c_body(acc_ref, den_ref, b2_ref, out_ref):
    den = den_ref[0, :, 0:1] + den_ref[1, :, 0:1]
    chunks = []
    for q in range(8):
        op = acc_ref[0, q] + acc_ref[1, q]
        op = op / (den + 1e-16) + b2_ref[0, q * 64:(q + 1) * 64]
        chunks.append(op)
    out_ref[...] = jnp.concatenate(chunks, axis=1)


def _tc_c(acc2, den2, b2):
    return pl.pallas_call(
        _tc_c_body,
        grid=(GRID,),
        in_specs=[
            pl.BlockSpec((2, 8, BN, 64), lambda i: (0, 0, i, 0)),
            pl.BlockSpec((2, BN, 16), lambda i: (0, i, 0)),
            pl.BlockSpec((1, 512), lambda i: (0, 0)),
        ],
        out_specs=pl.BlockSpec((BN, 512), lambda i: (i, 0)),
        out_shape=jax.ShapeDtypeStruct((N, 512), jnp.float32),
    )(acc2, den2, b2)


_MESH = plsc.VectorSubcoreMesh(core_axis_name="c", subcore_axis_name="s")


def _make_sc1(H):
    @functools.partial(
        pl.kernel,
        out_type=(
            jax.ShapeDtypeStruct((4, ROWS, 128), jnp.float32),
            jax.ShapeDtypeStruct((2, NP, 16), jnp.float32),
        ),
        mesh=_MESH,
        compiler_params=pltpu.CompilerParams(needs_layout_passes=False, use_tc_tiling_on_sc=False),
        scratch_types=[
            pltpu.VMEM((N, 8), jnp.float32),
            pltpu.VMEM((RPT, 128), jnp.int32),
            pltpu.VMEM((RPT, 128), jnp.int32),
            pltpu.VMEM((128, 16), jnp.float32),
            pltpu.VMEM((4, RPT, 128), jnp.float32),
            pltpu.VMEM((16,), jnp.float32),
            pltpu.VMEM((128, 16), jnp.float32),
            pltpu.VMEM_SHARED((NP, 16), jnp.float32),
        ],
    )
    def sc1(ab_hbm, src_hbm, dst_hbm, m_hbm, ex_hbm, den_hbm,
            ab_v, src_v, dst_v, exrow, exh, m_v, zbuf, den_sh):
        c = lax.axis_index("c")
        s = lax.axis_index("s")
        wid = c * 16 + s
        pltpu.sync_copy(ab_hbm, ab_v)
        pltpu.sync_copy(m_hbm, m_v)
        pltpu.sync_copy(src_hbm.at[pl.ds(wid * RPT, RPT)], src_v)
        pltpu.sync_copy(dst_hbm.at[pl.ds(wid * RPT, RPT)], dst_v)
        z16 = jnp.zeros((16,), jnp.float32)

        def _zb(j, carry):
            zbuf[j, :] = z16
            return carry

        lax.fori_loop(0, 128, _zb, 0)

        def _ze(j, carry):
            exrow[j, :] = z16
            return carry

        lax.fori_loop(0, 128, _ze, 0)
        for k in range(5):
            pltpu.sync_copy(zbuf, den_sh.at[pl.ds(s * STR + k * 128, 128)])
        plsc.subcore_barrier()

        iota16 = lax.iota(jnp.int32, 16)
        mreg = m_v[...]

        def _row(j, carry):
            row = wid * RPT + j
            for g in range(8):
                sidx = src_v[j, pl.ds(g * 16, 16)]
                didx = dst_v[j, pl.ds(g * 16, 16)]
                eid = row * 128 + g * 16 + iota16
                valid = eid < E
                for h in range(H):
                    va = plsc.load_gather(
                        ab_v, [sidx, jnp.full((16,), h, jnp.int32)])
                    vb = plsc.load_gather(
                        ab_v, [didx, jnp.full((16,), 4 + h, jnp.int32)])
                    al = va + vb
                    al = jnp.where(al > 0, al, al * NEG)
                    mb = jnp.full((16,), mreg[h], jnp.float32)
                    ex = jnp.exp(al - mb)
                    ex = jnp.where(valid, ex, 0.0)
                    exh[h, j, pl.ds(g * 16, 16)] = ex
                    plsc.store_scatter(
                        exrow,
                        [g * 16 + iota16, jnp.full((16,), h, jnp.int32)],
                        ex)
            pltpu.sync_copy(exrow, den_sh.at[dst_v.at[j]], add=True)
            return carry

        lax.fori_loop(0, RPT, _row, 0)
        for h in range(4):
            pltpu.sync_copy(exh.at[min(h, H - 1)],
                            ex_hbm.at[h, pl.ds(wid * RPT, RPT)])
        plsc.subcore_barrier()
        for k in range(5):
            pltpu.sync_copy(den_sh.at[pl.ds(s * STR + k * 128, 128)],
                            den_hbm.at[c, pl.ds(s * STR + k * 128, 128)])

    return sc1


_sc1_h4 = _make_sc1(4)
_sc1_h1 = _make_sc1(1)


@functools.partial(
    pl.kernel,
    out_type=jax.ShapeDtypeStruct((2, 8, NP, 64), jnp.float32),
    mesh=_MESH,
    compiler_params=pltpu.CompilerParams(needs_layout_passes=False, use_tc_tiling_on_sc=False),
    scratch_types=[
        pltpu.VMEM((RPH, 128), jnp.int32),
        pltpu.VMEM((RPH, 128), jnp.int32),
        pltpu.VMEM((128, 64), jnp.float32),
        pltpu.VMEM((128, 64), jnp.float32),
        pltpu.VMEM((128, 64), jnp.float32),
        pltpu.VMEM((128, 64), jnp.float32),
        pltpu.VMEM((RPH, 128), jnp.float32),
        pltpu.VMEM_SHARED((NP, 64), jnp.float32),
        pltpu.SemaphoreType.DMA,
        pltpu.SemaphoreType.DMA,
        pltpu.SemaphoreType.DMA,
        pltpu.SemaphoreType.DMA,
    ],
)
def _sc2(tbl_hbm, src_hbm, dst_hbm, ex_hbm, acc_hbm,
         dst_v, fidx, g0, g1, s0, s1, exb, acc_sh,
         sg0, sg1, ss0, ss1):
    c = lax.axis_index("c")
    s = lax.axis_index("s")
    base_row = c * (ROWS // 2) + s * RPH
    z16 = jnp.zeros((16,), jnp.float32)

    def _scale(gb_ref, sb_ref, j):
        def gb(g, carry):
            exv = exb[j, pl.ds(g * 16, 16)]
            for l in range(16):
                bc = jnp.full((16,), exv[l], jnp.float32)
                for sl in range(4):
                    sb_ref[g * 16 + l, pl.ds(sl * 16, 16)] = (
                        gb_ref[g * 16 + l, pl.ds(sl * 16, 16)] * bc)
            return carry
        lax.fori_loop(0, 8, gb, 0)

    pltpu.sync_copy(dst_hbm.at[pl.ds(base_row, RPH)], dst_v)

    def _pass(q, pcarry):
        def _zb(j, carry):
            for sl in range(4):
                s0[j, pl.ds(sl * 16, 16)] = z16
            return carry

        lax.fori_loop(0, 128, _zb, 0)
        for k in range(5):
            pltpu.sync_copy(s0, acc_sh.at[pl.ds(s * STR + k * 128, 128)])
        pltpu.sync_copy(ex_hbm.at[q // 2, pl.ds(base_row, RPH)], exb)
        pltpu.sync_copy(src_hbm.at[pl.ds(base_row, RPH)], fidx)

        def _fb(j, carry):
            for g in range(8):
                sv = fidx[j, pl.ds(g * 16, 16)]
                fidx[j, pl.ds(g * 16, 16)] = sv * 8 + q
            return carry

        lax.fori_loop(0, RPH, _fb, 0)
        plsc.subcore_barrier()

        pltpu.async_copy(tbl_hbm.at[fidx.at[0]], g0, sg0)
        pltpu.async_copy(tbl_hbm.at[fidx.at[1]], g1, sg1)
        NI = RPH // 2

        def _it(i, carry):
            j0 = 2 * i
            j1 = 2 * i + 1
            pltpu.make_async_copy(tbl_hbm.at[fidx.at[j0]], g0, sg0).wait()

            @pl.when(i > 0)
            def _():
                pltpu.make_async_copy(s0, acc_sh.at[dst_v.at[j0]], ss0).wait()

            _scale(g0, s0, j0)
            pltpu.async_copy(s0, acc_sh.at[dst_v.at[j0]], ss0, add=True)

            @pl.when(i < NI - 1)
            def _():
                pltpu.async_copy(tbl_hbm.at[fidx.at[j0 + 2]], g0, sg0)

            pltpu.make_async_copy(tbl_hbm.at[fidx.at[j1]], g1, sg1).wait()

            @pl.when(i > 0)
            def _():
                pltpu.make_async_copy(s1, acc_sh.at[dst_v.at[j1]], ss1).wait()

            _scale(g1, s1, j1)
            pltpu.async_copy(s1, acc_sh.at[dst_v.at[j1]], ss1, add=True)

            @pl.when(i < NI - 1)
            def _():
                pltpu.async_copy(tbl_hbm.at[fidx.at[j1 + 2]], g1, sg1)

            return carry

        lax.fori_loop(0, NI, _it, 0)
        pltpu.make_async_copy(s0, acc_sh.at[dst_v.at[0]], ss0).wait()
        pltpu.make_async_copy(s1, acc_sh.at[dst_v.at[1]], ss1).wait()
        plsc.subcore_barrier()
        for k in range(5):
            pltpu.sync_copy(acc_sh.at[pl.ds(s * STR + k * 128, 128)],
                            acc_hbm.at[c, q, pl.ds(s * STR + k * 128, 128)])
        plsc.subcore_barrier()
        return pcarry

    lax.fori_loop(0, 8, _pass, 0)


def _mvec(m_out, H):
    ms = m_out[0, :4] + m_out[0, 4:]
    ms = jnp.where(ms > 0, ms, ms * NEG)
    return jnp.zeros((16,), jnp.float32).at[:4].set(
        jnp.where(jnp.arange(4) < H, ms, 0.0))


def kernel(x, edge_index, W1, a_src1, a_dst1, b1, W2, a_src2, a_dst2, b2):
    src = edge_index[0]
    dst = edge_index[1]
    padz = (jnp.arange(E, EP, dtype=jnp.int32) * 61) % N
    src2d = jnp.concatenate([src, padz]).reshape(ROWS, 128)
    dst2d = jnp.concatenate([dst, padz]).reshape(ROWS, 128)

    a1s = a_src1.reshape(1, 512)
    a1d = a_dst1.reshape(1, 512)
    xw1, ab1, m1 = _tc_a(x, W1, a1s, a1d)
    ex1, den1 = _sc1_h4(ab1, src2d, dst2d, _mvec(m1, 4))
    acc1 = _sc2(xw1.reshape(N * 8, 64), src2d, dst2d, ex1)

    xw2, ab2, m2 = _tc_b(acc1, den1, b1.reshape(1, 512),
                         W2.reshape(4, 128, 512),
                         a_src2.reshape(1, 512), a_dst2.reshape(1, 512))
    ex2, den2 = _sc1_h1(ab2, src2d, dst2d, _mvec(m2, 1))
    acc2 = _sc2(xw2.reshape(N * 8, 64), src2d, dst2d, ex2)

    return _tc_c(acc2, den2, b2.reshape(1, 512))


# speedup vs baseline: 1.0926x; 1.0926x over previous
import functools

import jax
import jax.numpy as jnp
from jax import lax
from jax.experimental import pallas as pl
from jax.experimental.pallas import tpu as pltpu
import jax.experimental.pallas.tpu_sc as plsc

N = 10000
E = 160000
D_IN = 256
NEG = 0.2

LANES = 128
ROWS = 1280
EP = ROWS * LANES
NTILES = 32
RPT = ROWS // NTILES
RPH = ROWS // 2 // 16
NP = 10240
STR = NP // 16
BN = 200
GRID = N // BN


def _tc_a_body(x_ref, w_ref, asr_ref, adr_ref, xw_ref, ab_ref, m_ref):
    i = pl.program_id(0)
    xw = jnp.dot(x_ref[...], w_ref[...], preferred_element_type=jnp.float32)
    xw_ref[...] = xw
    s = (xw * asr_ref[...]).reshape(BN, 4, 128).sum(-1)
    d = (xw * adr_ref[...]).reshape(BN, 4, 128).sum(-1)
    ab = jnp.concatenate([s, d], axis=1)
    ab_ref[...] = ab
    cur = jnp.max(ab, axis=0, keepdims=True)

    @pl.when(i == 0)
    def _():
        m_ref[...] = cur

    @pl.when(i > 0)
    def _():
        m_ref[...] = jnp.maximum(m_ref[...], cur)


def _tc_a(x, W1, a1s, a1d):
    return pl.pallas_call(
        _tc_a_body,
        grid=(GRID,),
        in_specs=[
            pl.BlockSpec((BN, D_IN), lambda i: (i, 0)),
            pl.BlockSpec((D_IN, 512), lambda i: (0, 0)),
            pl.BlockSpec((1, 512), lambda i: (0, 0)),
            pl.BlockSpec((1, 512), lambda i: (0, 0)),
        ],
        out_specs=[
            pl.BlockSpec((BN, 512), lambda i: (i, 0)),
            pl.BlockSpec((BN, 8), lambda i: (i, 0)),
            pl.BlockSpec((1, 8), lambda i: (0, 0)),
        ],
        out_shape=[
            jax.ShapeDtypeStruct((N, 512), jnp.float32),
            jax.ShapeDtypeStruct((N, 8), jnp.float32),
            jax.ShapeDtypeStruct((1, 8), jnp.float32),
        ],
    )(x, W1, a1s, a1d)


def _tc_b_body(acc_ref, den_ref, b1_ref, w2_ref, a2s_ref, a2d_ref,
               xw_ref, ab_ref, m_ref):
    i = pl.program_id(0)
    den = den_ref[0] + den_ref[1]
    xw = jnp.zeros((BN, 512), jnp.float32)
    for p in range(4):
        hp = acc_ref[0, p] + acc_ref[1, p]
        hp = hp / (den[:, p:p + 1] + 1e-16) + b1_ref[0, p * 128:(p + 1) * 128]
        hp = jnp.maximum(hp, 0.0)
        xw = xw + jnp.dot(hp, w2_ref[p], preferred_element_type=jnp.float32)
    xw_ref[...] = xw
    s = (xw * a2s_ref[...]).sum(-1, keepdims=True)
    d = (xw * a2d_ref[...]).sum(-1, keepdims=True)
    z = jnp.zeros((BN, 3), jnp.float32)
    ab = jnp.concatenate([s, z, d, z], axis=1)
    ab_ref[...] = ab
    cur = jnp.max(ab, axis=0, keepdims=True)

    @pl.when(i == 0)
    def _():
        m_ref[...] = cur

    @pl.when(i > 0)
    def _():
        m_ref[...] = jnp.maximum(m_ref[...], cur)


def _tc_b(acc1, den1, b1, W2r, a2s, a2d):
    return pl.pallas_call(
        _tc_b_body,
        grid=(GRID,),
        in_specs=[
            pl.BlockSpec((2, 4, BN, 128), lambda i: (0, 0, i, 0)),
            pl.BlockSpec((2, BN, 16), lambda i: (0, i, 0)),
            pl.BlockSpec((1, 512), lambda i: (0, 0)),
            pl.BlockSpec((4, 128, 512), lambda i: (0, 0, 0)),
            pl.BlockSpec((1, 512), lambda i: (0, 0)),
            pl.BlockSpec((1, 512), lambda i: (0, 0)),
        ],
        out_specs=[
            pl.BlockSpec((BN, 512), lambda i: (i, 0)),
            pl.BlockSpec((BN, 8), lambda i: (i, 0)),
            pl.BlockSpec((1, 8), lambda i: (0, 0)),
        ],
        out_shape=[
            jax.ShapeDtypeStruct((N, 512), jnp.float32),
            jax.ShapeDtypeStruct((N, 8), jnp.float32),
            jax.ShapeDtypeStruct((1, 8), jnp.float32),
        ],
    )(acc1, den1, b1, W2r, a2s, a2d)


def _tc_c_body(acc_ref, den_ref, b2_ref, out_ref):
    den = den_ref[0, :, 0:1] + den_ref[1, :, 0:1]
    chunks = []
    for p in range(4):
        op = acc_ref[0, p] + acc_ref[1, p]
        op = op / (den + 1e-16) + b2_ref[0, p * 128:(p + 1) * 128]
        chunks.append(op)
    out_ref[...] = jnp.concatenate(chunks, axis=1)


def _tc_c(acc2, den2, b2):
    return pl.pallas_call(
        _tc_c_body,
        grid=(GRID,),
        in_specs=[
            pl.BlockSpec((2, 4, BN, 128), lambda i: (0, 0, i, 0)),
            pl.BlockSpec((2, BN, 16), lambda i: (0, i, 0)),
            pl.BlockSpec((1, 512), lambda i: (0, 0)),
        ],
        out_specs=pl.BlockSpec((BN, 512), lambda i: (i, 0)),
        out_shape=jax.ShapeDtypeStruct((N, 512), jnp.float32),
    )(acc2, den2, b2)


_MESH = plsc.VectorSubcoreMesh(core_axis_name="c", subcore_axis_name="s")


def _make_sc1(H):
    @functools.partial(
        pl.kernel,
        out_type=(
            jax.ShapeDtypeStruct((4, ROWS, 128), jnp.float32),
            jax.ShapeDtypeStruct((2, NP, 16), jnp.float32),
        ),
        mesh=_MESH,
        compiler_params=pltpu.CompilerParams(needs_layout_passes=False, use_tc_tiling_on_sc=False),
        scratch_types=[
            pltpu.VMEM((N, 8), jnp.float32),
            pltpu.VMEM((RPT, 128), jnp.int32),
            pltpu.VMEM((RPT, 128), jnp.int32),
            pltpu.VMEM((128, 16), jnp.float32),
            pltpu.VMEM((4, RPT, 128), jnp.float32),
            pltpu.VMEM((16,), jnp.float32),
            pltpu.VMEM((128, 16), jnp.float32),
            pltpu.VMEM_SHARED((NP, 16), jnp.float32),
        ],
    )
    def sc1(ab_hbm, src_hbm, dst_hbm, m_hbm, ex_hbm, den_hbm,
            ab_v, src_v, dst_v, exrow, exh, m_v, zbuf, den_sh):
        c = lax.axis_index("c")
        s = lax.axis_index("s")
        wid = c * 16 + s
        pltpu.sync_copy(ab_hbm, ab_v)
        pltpu.sync_copy(m_hbm, m_v)
        pltpu.sync_copy(src_hbm.at[pl.ds(wid * RPT, RPT)], src_v)
        pltpu.sync_copy(dst_hbm.at[pl.ds(wid * RPT, RPT)], dst_v)
        z16 = jnp.zeros((16,), jnp.float32)

        def _zb(j, carry):
            zbuf[j, :] = z16
            return carry

        lax.fori_loop(0, 128, _zb, 0)

        def _ze(j, carry):
            exrow[j, :] = z16
            return carry

        lax.fori_loop(0, 128, _ze, 0)
        for k in range(5):
            pltpu.sync_copy(zbuf, den_sh.at[pl.ds(s * STR + k * 128, 128)])
        plsc.subcore_barrier()

        iota16 = lax.iota(jnp.int32, 16)
        mreg = m_v[...]

        def _row(j, carry):
            row = wid * RPT + j
            for g in range(8):
                sidx = src_v[j, pl.ds(g * 16, 16)]
                didx = dst_v[j, pl.ds(g * 16, 16)]
                eid = row * 128 + g * 16 + iota16
                valid = eid < E
                for h in range(H):
                    va = plsc.load_gather(
                        ab_v, [sidx, jnp.full((16,), h, jnp.int32)])
                    vb = plsc.load_gather(
                        ab_v, [didx, jnp.full((16,), 4 + h, jnp.int32)])
                    al = va + vb
                    al = jnp.where(al > 0, al, al * NEG)
                    mb = jnp.full((16,), mreg[h], jnp.float32)
                    ex = jnp.exp(al - mb)
                    ex = jnp.where(valid, ex, 0.0)
                    exh[h, j, pl.ds(g * 16, 16)] = ex
                    plsc.store_scatter(
                        exrow,
                        [g * 16 + iota16, jnp.full((16,), h, jnp.int32)],
                        ex)
            pltpu.sync_copy(exrow, den_sh.at[dst_v.at[j]], add=True)
            return carry

        lax.fori_loop(0, RPT, _row, 0)
        for h in range(4):
            pltpu.sync_copy(exh.at[min(h, H - 1)],
                            ex_hbm.at[h, pl.ds(wid * RPT, RPT)])
        plsc.subcore_barrier()
        for k in range(5):
            pltpu.sync_copy(den_sh.at[pl.ds(s * STR + k * 128, 128)],
                            den_hbm.at[c, pl.ds(s * STR + k * 128, 128)])

    return sc1


_sc1_h4 = _make_sc1(4)
_sc1_h1 = _make_sc1(1)


@functools.partial(
    pl.kernel,
    out_type=jax.ShapeDtypeStruct((2, 4, NP, 128), jnp.float32),
    mesh=_MESH,
    compiler_params=pltpu.CompilerParams(needs_layout_passes=False, use_tc_tiling_on_sc=False),
    scratch_types=[
        pltpu.VMEM((RPH, 128), jnp.int32),
        pltpu.VMEM((RPH, 128), jnp.int32),
        pltpu.VMEM((128, 128), jnp.float32),
        pltpu.VMEM((128, 128), jnp.float32),
        pltpu.VMEM((RPH, 128), jnp.float32),
        pltpu.VMEM_SHARED((NP, 128), jnp.float32),
        pltpu.SemaphoreType.DMA,
        pltpu.SemaphoreType.DMA,
        pltpu.SemaphoreType.DMA,
        pltpu.SemaphoreType.DMA,
    ],
)
def _sc2(tbl_hbm, src_hbm, dst_hbm, ex_hbm, acc_hbm,
         dst_v, fidx, rows_a, rows_b, exb, acc_sh,
         sga, sgb, ssa, ssb):
    c = lax.axis_index("c")
    s = lax.axis_index("s")
    base_row = c * (ROWS // 2) + s * RPH
    HALF = RPH // 2
    z16 = jnp.zeros((16,), jnp.float32)

    def _scale(rb, j):
        def gb(g, carry):
            exv = exb[j, pl.ds(g * 16, 16)]
            for l in range(16):
                bc = jnp.full((16,), exv[l], jnp.float32)
                for sl in range(8):
                    rb[g * 16 + l, pl.ds(sl * 16, 16)] = (
                        rb[g * 16 + l, pl.ds(sl * 16, 16)] * bc)
            return carry
        lax.fori_loop(0, 8, gb, 0)

    pltpu.sync_copy(dst_hbm.at[pl.ds(base_row, RPH)], dst_v)

    def _pass(p, pcarry):
        def _zb(j, carry):
            for sl in range(8):
                rows_a[j, pl.ds(sl * 16, 16)] = z16
            return carry

        lax.fori_loop(0, 128, _zb, 0)
        for k in range(5):
            pltpu.sync_copy(rows_a, acc_sh.at[pl.ds(s * STR + k * 128, 128)])
        pltpu.sync_copy(ex_hbm.at[p, pl.ds(base_row, RPH)], exb)
        pltpu.sync_copy(src_hbm.at[pl.ds(base_row, RPH)], fidx)

        def _fb(j, carry):
            for g in range(8):
                sv = fidx[j, pl.ds(g * 16, 16)]
                fidx[j, pl.ds(g * 16, 16)] = sv * 4 + p
            return carry

        lax.fori_loop(0, RPH, _fb, 0)
        plsc.subcore_barrier()

        pltpu.async_copy(tbl_hbm.at[fidx.at[0]], rows_a, sga)

        def _it(i, carry):
            j0 = 2 * i
            j1 = 2 * i + 1
            pltpu.make_async_copy(tbl_hbm.at[fidx.at[j0]], rows_a, sga).wait()

            @pl.when(i > 0)
            def _():
                pltpu.make_async_copy(
                    rows_b, acc_sh.at[dst_v.at[j1]], ssb).wait()

            pltpu.async_copy(tbl_hbm.at[fidx.at[j1]], rows_b, sgb)
            _scale(rows_a, j0)
            pltpu.async_copy(rows_a, acc_sh.at[dst_v.at[j0]], ssa, add=True)
            pltpu.make_async_copy(tbl_hbm.at[fidx.at[j1]], rows_b, sgb).wait()
            _scale(rows_b, j1)
            pltpu.make_async_copy(rows_a, acc_sh.at[dst_v.at[j0]], ssa).wait()

            @pl.when(i < HALF - 1)
            def _():
                pltpu.async_copy(tbl_hbm.at[fidx.at[j0 + 2]], rows_a, sga)

            pltpu.async_copy(rows_b, acc_sh.at[dst_v.at[j1]], ssb, add=True)
            return carry

        lax.fori_loop(0, HALF, _it, 0)
        pltpu.make_async_copy(rows_b, acc_sh.at[dst_v.at[0]], ssb).wait()
        plsc.subcore_barrier()
        for k in range(5):
            pltpu.sync_copy(acc_sh.at[pl.ds(s * STR + k * 128, 128)],
                            acc_hbm.at[c, p, pl.ds(s * STR + k * 128, 128)])
        plsc.subcore_barrier()
        return pcarry

    lax.fori_loop(0, 4, _pass, 0)


def _mvec(m_out, H):
    ms = m_out[0, :4] + m_out[0, 4:]
    ms = jnp.where(ms > 0, ms, ms * NEG)
    return jnp.zeros((16,), jnp.float32).at[:4].set(
        jnp.where(jnp.arange(4) < H, ms, 0.0))


def kernel(x, edge_index, W1, a_src1, a_dst1, b1, W2, a_src2, a_dst2, b2):
    src = edge_index[0]
    dst = edge_index[1]
    padz = (jnp.arange(E, EP, dtype=jnp.int32) * 61) % N
    src2d = jnp.concatenate([src, padz]).reshape(ROWS, 128)
    dst2d = jnp.concatenate([dst, padz]).reshape(ROWS, 128)

    a1s = a_src1.reshape(1, 512)
    a1d = a_dst1.reshape(1, 512)
    xw1, ab1, m1 = _tc_a(x, W1, a1s, a1d)
    ex1, den1 = _sc1_h4(ab1, src2d, dst2d, _mvec(m1, 4))
    acc1 = _sc2(xw1.reshape(N * 4, 128), src2d, dst2d, ex1)

    xw2, ab2, m2 = _tc_b(acc1, den1, b1.reshape(1, 512),
                         W2.reshape(4, 128, 512),
                         a_src2.reshape(1, 512), a_dst2.reshape(1, 512))
    ex2, den2 = _sc1_h1(ab2, src2d, dst2d, _mvec(m2, 1))
    acc2 = _sc2(xw2.reshape(N * 4, 128), src2d, dst2d, ex2)

    return _tc_c(acc2, den2, b2.reshape(1, 512))


# speedup vs baseline: 1.1437x; 1.0467x over previous
import functools

import jax
import jax.numpy as jnp
from jax import lax
from jax.experimental import pallas as pl
from jax.experimental.pallas import tpu as pltpu
import jax.experimental.pallas.tpu_sc as plsc

N = 10000
E = 160000
D_IN = 256
NEG = 0.2

LANES = 128
ROWS = 1280
EP = ROWS * LANES
NTILES = 32
RPT = ROWS // NTILES
RPH = ROWS // 2 // 16
NP = 10240
STR = NP // 16
BN = 400
GRID = N // BN


def _tc_a_body(x_ref, w_ref, asr_ref, adr_ref, xw_ref, ab_ref, m_ref):
    i = pl.program_id(0)
    xw = jnp.dot(x_ref[...], w_ref[...], preferred_element_type=jnp.float32)
    xw_ref[...] = xw
    s = (xw * asr_ref[...]).reshape(BN, 4, 128).sum(-1)
    d = (xw * adr_ref[...]).reshape(BN, 4, 128).sum(-1)
    ab = jnp.concatenate([s, d], axis=1)
    ab_ref[...] = ab
    cur = jnp.max(ab, axis=0, keepdims=True)

    @pl.when(i == 0)
    def _():
        m_ref[...] = cur

    @pl.when(i > 0)
    def _():
        m_ref[...] = jnp.maximum(m_ref[...], cur)


def _tc_a(x, W1, a1s, a1d):
    return pl.pallas_call(
        _tc_a_body,
        grid=(GRID,),
        in_specs=[
            pl.BlockSpec((BN, D_IN), lambda i: (i, 0)),
            pl.BlockSpec((D_IN, 512), lambda i: (0, 0)),
            pl.BlockSpec((1, 512), lambda i: (0, 0)),
            pl.BlockSpec((1, 512), lambda i: (0, 0)),
        ],
        out_specs=[
            pl.BlockSpec((BN, 512), lambda i: (i, 0)),
            pl.BlockSpec((BN, 8), lambda i: (i, 0)),
            pl.BlockSpec((1, 8), lambda i: (0, 0)),
        ],
        out_shape=[
            jax.ShapeDtypeStruct((N, 512), jnp.float32),
            jax.ShapeDtypeStruct((N, 8), jnp.float32),
            jax.ShapeDtypeStruct((1, 8), jnp.float32),
        ],
    )(x, W1, a1s, a1d)


def _tc_b_body(acc_ref, den_ref, b1_ref, w2_ref, a2s_ref, a2d_ref,
               xw_ref, ab_ref, m_ref):
    i = pl.program_id(0)
    den = den_ref[0] + den_ref[1]
    xw = jnp.zeros((BN, 512), jnp.float32)
    for p in range(4):
        hp = acc_ref[0, p] + acc_ref[1, p]
        hp = hp / (den[:, p:p + 1] + 1e-16) + b1_ref[0, p * 128:(p + 1) * 128]
        hp = jnp.maximum(hp, 0.0)
        xw = xw + jnp.dot(hp, w2_ref[p], preferred_element_type=jnp.float32)
    xw_ref[...] = xw
    s = (xw * a2s_ref[...]).sum(-1, keepdims=True)
    d = (xw * a2d_ref[...]).sum(-1, keepdims=True)
    z = jnp.zeros((BN, 3), jnp.float32)
    ab = jnp.concatenate([s, z, d, z], axis=1)
    ab_ref[...] = ab
    cur = jnp.max(ab, axis=0, keepdims=True)

    @pl.when(i == 0)
    def _():
        m_ref[...] = cur

    @pl.when(i > 0)
    def _():
        m_ref[...] = jnp.maximum(m_ref[...], cur)


def _tc_b(acc1, den1, b1, W2r, a2s, a2d):
    return pl.pallas_call(
        _tc_b_body,
        grid=(GRID,),
        in_specs=[
            pl.BlockSpec((2, 4, BN, 128), lambda i: (0, 0, i, 0)),
            pl.BlockSpec((2, BN, 16), lambda i: (0, i, 0)),
            pl.BlockSpec((1, 512), lambda i: (0, 0)),
            pl.BlockSpec((4, 128, 512), lambda i: (0, 0, 0)),
            pl.BlockSpec((1, 512), lambda i: (0, 0)),
            pl.BlockSpec((1, 512), lambda i: (0, 0)),
        ],
        out_specs=[
            pl.BlockSpec((BN, 512), lambda i: (i, 0)),
            pl.BlockSpec((BN, 8), lambda i: (i, 0)),
            pl.BlockSpec((1, 8), lambda i: (0, 0)),
        ],
        out_shape=[
            jax.ShapeDtypeStruct((N, 512), jnp.float32),
            jax.ShapeDtypeStruct((N, 8), jnp.float32),
            jax.ShapeDtypeStruct((1, 8), jnp.float32),
        ],
    )(acc1, den1, b1, W2r, a2s, a2d)


def _tc_c_body(acc_ref, den_ref, b2_ref, out_ref):
    den = den_ref[0, :, 0:1] + den_ref[1, :, 0:1]
    chunks = []
    for p in range(4):
        op = acc_ref[0, p] + acc_ref[1, p]
        op = op / (den + 1e-16) + b2_ref[0, p * 128:(p + 1) * 128]
        chunks.append(op)
    out_ref[...] = jnp.concatenate(chunks, axis=1)


def _tc_c(acc2, den2, b2):
    return pl.pallas_call(
        _tc_c_body,
        grid=(GRID,),
        in_specs=[
            pl.BlockSpec((2, 4, BN, 128), lambda i: (0, 0, i, 0)),
            pl.BlockSpec((2, BN, 16), lambda i: (0, i, 0)),
            pl.BlockSpec((1, 512), lambda i: (0, 0)),
        ],
        out_specs=pl.BlockSpec((BN, 512), lambda i: (i, 0)),
        out_shape=jax.ShapeDtypeStruct((N, 512), jnp.float32),
    )(acc2, den2, b2)


_MESH = plsc.VectorSubcoreMesh(core_axis_name="c", subcore_axis_name="s")


def _make_sc1(H):
    @functools.partial(
        pl.kernel,
        out_type=(
            jax.ShapeDtypeStruct((4, ROWS, 128), jnp.float32),
            jax.ShapeDtypeStruct((2, NP, 16), jnp.float32),
        ),
        mesh=_MESH,
        compiler_params=pltpu.CompilerParams(needs_layout_passes=False, use_tc_tiling_on_sc=False),
        scratch_types=[
            pltpu.VMEM((N, 8), jnp.float32),
            pltpu.VMEM((RPT, 128), jnp.int32),
            pltpu.VMEM((RPT, 128), jnp.int32),
            pltpu.VMEM((128, 16), jnp.float32),
            pltpu.VMEM((4, RPT, 128), jnp.float32),
            pltpu.VMEM((16,), jnp.float32),
            pltpu.VMEM((128, 16), jnp.float32),
            pltpu.VMEM_SHARED((NP, 16), jnp.float32),
        ],
    )
    def sc1(ab_hbm, src_hbm, dst_hbm, m_hbm, ex_hbm, den_hbm,
            ab_v, src_v, dst_v, exrow, exh, m_v, zbuf, den_sh):
        c = lax.axis_index("c")
        s = lax.axis_index("s")
        wid = c * 16 + s
        pltpu.sync_copy(ab_hbm, ab_v)
        pltpu.sync_copy(m_hbm, m_v)
        pltpu.sync_copy(src_hbm.at[pl.ds(wid * RPT, RPT)], src_v)
        pltpu.sync_copy(dst_hbm.at[pl.ds(wid * RPT, RPT)], dst_v)
        z16 = jnp.zeros((16,), jnp.float32)

        def _zb(j, carry):
            zbuf[j, :] = z16
            return carry

        lax.fori_loop(0, 128, _zb, 0)

        def _ze(j, carry):
            exrow[j, :] = z16
            return carry

        lax.fori_loop(0, 128, _ze, 0)
        for k in range(5):
            pltpu.sync_copy(zbuf, den_sh.at[pl.ds(s * STR + k * 128, 128)])
        plsc.subcore_barrier()

        iota16 = lax.iota(jnp.int32, 16)
        mreg = m_v[...]

        def _row(j, carry):
            row = wid * RPT + j
            for g in range(8):
                sidx = src_v[j, pl.ds(g * 16, 16)]
                didx = dst_v[j, pl.ds(g * 16, 16)]
                eid = row * 128 + g * 16 + iota16
                valid = eid < E
                for h in range(H):
                    va = plsc.load_gather(
                        ab_v, [sidx, jnp.full((16,), h, jnp.int32)])
                    vb = plsc.load_gather(
                        ab_v, [didx, jnp.full((16,), 4 + h, jnp.int32)])
                    al = va + vb
                    al = jnp.where(al > 0, al, al * NEG)
                    mb = jnp.full((16,), mreg[h], jnp.float32)
                    ex = jnp.exp(al - mb)
                    ex = jnp.where(valid, ex, 0.0)
                    exh[h, j, pl.ds(g * 16, 16)] = ex
                    plsc.store_scatter(
                        exrow,
                        [g * 16 + iota16, jnp.full((16,), h, jnp.int32)],
                        ex)
            pltpu.sync_copy(exrow, den_sh.at[dst_v.at[j]], add=True)
            return carry

        lax.fori_loop(0, RPT, _row, 0)
        for h in range(4):
            pltpu.sync_copy(exh.at[min(h, H - 1)],
                            ex_hbm.at[h, pl.ds(wid * RPT, RPT)])
        plsc.subcore_barrier()
        for k in range(5):
            pltpu.sync_copy(den_sh.at[pl.ds(s * STR + k * 128, 128)],
                            den_hbm.at[c, pl.ds(s * STR + k * 128, 128)])

    return sc1


_sc1_h4 = _make_sc1(4)
_sc1_h1 = _make_sc1(1)


@functools.partial(
    pl.kernel,
    out_type=jax.ShapeDtypeStruct((2, 4, NP, 128), jnp.float32),
    mesh=_MESH,
    compiler_params=pltpu.CompilerParams(needs_layout_passes=False, use_tc_tiling_on_sc=False),
    scratch_types=[
        pltpu.VMEM((RPH, 128), jnp.int32),
        pltpu.VMEM((RPH, 128), jnp.int32),
        pltpu.VMEM((128, 128), jnp.float32),
        pltpu.VMEM((128, 128), jnp.float32),
        pltpu.VMEM((RPH, 128), jnp.float32),
        pltpu.VMEM_SHARED((NP, 128), jnp.float32),
        pltpu.SemaphoreType.DMA,
        pltpu.SemaphoreType.DMA,
        pltpu.SemaphoreType.DMA,
        pltpu.SemaphoreType.DMA,
    ],
)
def _sc2(tbl_hbm, src_hbm, dst_hbm, ex_hbm, acc_hbm,
         dst_v, fidx, rows_a, rows_b, exb, acc_sh,
         sga, sgb, ssa, ssb):
    c = lax.axis_index("c")
    s = lax.axis_index("s")
    base_row = c * (ROWS // 2) + s * RPH
    HALF = RPH // 2
    z16 = jnp.zeros((16,), jnp.float32)

    def _scale(rb, j):
        def gb(g, carry):
            exv = exb[j, pl.ds(g * 16, 16)]
            for l in range(16):
                bc = jnp.full((16,), exv[l], jnp.float32)
                for sl in range(8):
                    rb[g * 16 + l, pl.ds(sl * 16, 16)] = (
                        rb[g * 16 + l, pl.ds(sl * 16, 16)] * bc)
            return carry
        lax.fori_loop(0, 8, gb, 0)

    pltpu.sync_copy(dst_hbm.at[pl.ds(base_row, RPH)], dst_v)

    def _pass(p, pcarry):
        def _zb(j, carry):
            for sl in range(8):
                rows_a[j, pl.ds(sl * 16, 16)] = z16
            return carry

        lax.fori_loop(0, 128, _zb, 0)
        for k in range(5):
            pltpu.sync_copy(rows_a, acc_sh.at[pl.ds(s * STR + k * 128, 128)])
        pltpu.sync_copy(ex_hbm.at[p, pl.ds(base_row, RPH)], exb)
        pltpu.sync_copy(src_hbm.at[pl.ds(base_row, RPH)], fidx)

        def _fb(j, carry):
            for g in range(8):
                sv = fidx[j, pl.ds(g * 16, 16)]
                fidx[j, pl.ds(g * 16, 16)] = sv * 4 + p
            return carry

        lax.fori_loop(0, RPH, _fb, 0)
        plsc.subcore_barrier()

        pltpu.async_copy(tbl_hbm.at[fidx.at[0]], rows_a, sga)

        def _it(i, carry):
            j0 = 2 * i
            j1 = 2 * i + 1
            pltpu.make_async_copy(tbl_hbm.at[fidx.at[j0]], rows_a, sga).wait()

            @pl.when(i > 0)
            def _():
                pltpu.make_async_copy(
                    rows_b, acc_sh.at[dst_v.at[j1]], ssb).wait()

            pltpu.async_copy(tbl_hbm.at[fidx.at[j1]], rows_b, sgb)
            _scale(rows_a, j0)
            pltpu.async_copy(rows_a, acc_sh.at[dst_v.at[j0]], ssa, add=True)
            pltpu.make_async_copy(tbl_hbm.at[fidx.at[j1]], rows_b, sgb).wait()
            _scale(rows_b, j1)
            pltpu.make_async_copy(rows_a, acc_sh.at[dst_v.at[j0]], ssa).wait()

            @pl.when(i < HALF - 1)
            def _():
                pltpu.async_copy(tbl_hbm.at[fidx.at[j0 + 2]], rows_a, sga)

            pltpu.async_copy(rows_b, acc_sh.at[dst_v.at[j1]], ssb, add=True)
            return carry

        lax.fori_loop(0, HALF, _it, 0)
        pltpu.make_async_copy(rows_b, acc_sh.at[dst_v.at[0]], ssb).wait()
        plsc.subcore_barrier()
        for k in range(5):
            pltpu.sync_copy(acc_sh.at[pl.ds(s * STR + k * 128, 128)],
                            acc_hbm.at[c, p, pl.ds(s * STR + k * 128, 128)])
        plsc.subcore_barrier()
        return pcarry

    lax.fori_loop(0, 4, _pass, 0)


def _mvec(m_out, H):
    ms = m_out[0, :4] + m_out[0, 4:]
    ms = jnp.where(ms > 0, ms, ms * NEG)
    return jnp.zeros((16,), jnp.float32).at[:4].set(
        jnp.where(jnp.arange(4) < H, ms, 0.0))


def kernel(x, edge_index, W1, a_src1, a_dst1, b1, W2, a_src2, a_dst2, b2):
    src = edge_index[0]
    dst = edge_index[1]
    padz = (jnp.arange(E, EP, dtype=jnp.int32) * 61) % N
    src2d = jnp.concatenate([src, padz]).reshape(ROWS, 128)
    dst2d = jnp.concatenate([dst, padz]).reshape(ROWS, 128)

    a1s = a_src1.reshape(1, 512)
    a1d = a_dst1.reshape(1, 512)
    xw1, ab1, m1 = _tc_a(x, W1, a1s, a1d)
    ex1, den1 = _sc1_h4(ab1, src2d, dst2d, _mvec(m1, 4))
    acc1 = _sc2(xw1.reshape(N * 4, 128), src2d, dst2d, ex1)

    xw2, ab2, m2 = _tc_b(acc1, den1, b1.reshape(1, 512),
                         W2.reshape(4, 128, 512),
                         a_src2.reshape(1, 512), a_dst2.reshape(1, 512))
    ex2, den2 = _sc1_h1(ab2, src2d, dst2d, _mvec(m2, 1))
    acc2 = _sc2(xw2.reshape(N * 4, 128), src2d, dst2d, ex2)

    return _tc_c(acc2, den2, b2.reshape(1, 512))


# speedup vs baseline: 1.1712x; 1.0241x over previous
import functools

import jax
import jax.numpy as jnp
from jax import lax
from jax.experimental import pallas as pl
from jax.experimental.pallas import tpu as pltpu
import jax.experimental.pallas.tpu_sc as plsc

N = 10000
E = 160000
D_IN = 256
NEG = 0.2

LANES = 128
ROWS = 1280
EP = ROWS * LANES
NTILES = 32
RPT = ROWS // NTILES
RPH = ROWS // 2 // 16
NP = 10240
STR = NP // 16
BN = 1000
GRID = N // BN


def _tc_a_body(x_ref, w_ref, asr_ref, adr_ref, xw_ref, ab_ref, m_ref):
    i = pl.program_id(0)
    xw = jnp.dot(x_ref[...], w_ref[...], preferred_element_type=jnp.float32)
    xw_ref[...] = xw
    s = (xw * asr_ref[...]).reshape(BN, 4, 128).sum(-1)
    d = (xw * adr_ref[...]).reshape(BN, 4, 128).sum(-1)
    ab = jnp.concatenate([s, d], axis=1)
    ab_ref[...] = ab
    cur = jnp.max(ab, axis=0, keepdims=True)

    @pl.when(i == 0)
    def _():
        m_ref[...] = cur

    @pl.when(i > 0)
    def _():
        m_ref[...] = jnp.maximum(m_ref[...], cur)


def _tc_a(x, W1, a1s, a1d):
    return pl.pallas_call(
        _tc_a_body,
        grid=(GRID,),
        in_specs=[
            pl.BlockSpec((BN, D_IN), lambda i: (i, 0)),
            pl.BlockSpec((D_IN, 512), lambda i: (0, 0)),
            pl.BlockSpec((1, 512), lambda i: (0, 0)),
            pl.BlockSpec((1, 512), lambda i: (0, 0)),
        ],
        out_specs=[
            pl.BlockSpec((BN, 512), lambda i: (i, 0)),
            pl.BlockSpec((BN, 8), lambda i: (i, 0)),
            pl.BlockSpec((1, 8), lambda i: (0, 0)),
        ],
        out_shape=[
            jax.ShapeDtypeStruct((N, 512), jnp.float32),
            jax.ShapeDtypeStruct((N, 8), jnp.float32),
            jax.ShapeDtypeStruct((1, 8), jnp.float32),
        ],
    )(x, W1, a1s, a1d)


def _tc_b_body(acc_ref, den_ref, b1_ref, w2_ref, a2s_ref, a2d_ref,
               xw_ref, ab_ref, m_ref):
    i = pl.program_id(0)
    den = den_ref[0] + den_ref[1]
    xw = jnp.zeros((BN, 512), jnp.float32)
    for p in range(4):
        hp = acc_ref[0, p] + acc_ref[1, p]
        hp = hp / (den[:, p:p + 1] + 1e-16) + b1_ref[0, p * 128:(p + 1) * 128]
        hp = jnp.maximum(hp, 0.0)
        xw = xw + jnp.dot(hp, w2_ref[p], preferred_element_type=jnp.float32)
    xw_ref[...] = xw
    s = (xw * a2s_ref[...]).sum(-1, keepdims=True)
    d = (xw * a2d_ref[...]).sum(-1, keepdims=True)
    z = jnp.zeros((BN, 3), jnp.float32)
    ab = jnp.concatenate([s, z, d, z], axis=1)
    ab_ref[...] = ab
    cur = jnp.max(ab, axis=0, keepdims=True)

    @pl.when(i == 0)
    def _():
        m_ref[...] = cur

    @pl.when(i > 0)
    def _():
        m_ref[...] = jnp.maximum(m_ref[...], cur)


def _tc_b(acc1, den1, b1, W2r, a2s, a2d):
    return pl.pallas_call(
        _tc_b_body,
        grid=(GRID,),
        in_specs=[
            pl.BlockSpec((2, 4, BN, 128), lambda i: (0, 0, i, 0)),
            pl.BlockSpec((2, BN, 16), lambda i: (0, i, 0)),
            pl.BlockSpec((1, 512), lambda i: (0, 0)),
            pl.BlockSpec((4, 128, 512), lambda i: (0, 0, 0)),
            pl.BlockSpec((1, 512), lambda i: (0, 0)),
            pl.BlockSpec((1, 512), lambda i: (0, 0)),
        ],
        out_specs=[
            pl.BlockSpec((BN, 512), lambda i: (i, 0)),
            pl.BlockSpec((BN, 8), lambda i: (i, 0)),
            pl.BlockSpec((1, 8), lambda i: (0, 0)),
        ],
        out_shape=[
            jax.ShapeDtypeStruct((N, 512), jnp.float32),
            jax.ShapeDtypeStruct((N, 8), jnp.float32),
            jax.ShapeDtypeStruct((1, 8), jnp.float32),
        ],
    )(acc1, den1, b1, W2r, a2s, a2d)


def _tc_c_body(acc_ref, den_ref, b2_ref, out_ref):
    den = den_ref[0, :, 0:1] + den_ref[1, :, 0:1]
    chunks = []
    for p in range(4):
        op = acc_ref[0, p] + acc_ref[1, p]
        op = op / (den + 1e-16) + b2_ref[0, p * 128:(p + 1) * 128]
        chunks.append(op)
    out_ref[...] = jnp.concatenate(chunks, axis=1)


def _tc_c(acc2, den2, b2):
    return pl.pallas_call(
        _tc_c_body,
        grid=(GRID,),
        in_specs=[
            pl.BlockSpec((2, 4, BN, 128), lambda i: (0, 0, i, 0)),
            pl.BlockSpec((2, BN, 16), lambda i: (0, i, 0)),
            pl.BlockSpec((1, 512), lambda i: (0, 0)),
        ],
        out_specs=pl.BlockSpec((BN, 512), lambda i: (i, 0)),
        out_shape=jax.ShapeDtypeStruct((N, 512), jnp.float32),
    )(acc2, den2, b2)


_MESH = plsc.VectorSubcoreMesh(core_axis_name="c", subcore_axis_name="s")


def _make_sc1(H):
    @functools.partial(
        pl.kernel,
        out_type=(
            jax.ShapeDtypeStruct((4, ROWS, 128), jnp.float32),
            jax.ShapeDtypeStruct((2, NP, 16), jnp.float32),
        ),
        mesh=_MESH,
        compiler_params=pltpu.CompilerParams(needs_layout_passes=False, use_tc_tiling_on_sc=False),
        scratch_types=[
            pltpu.VMEM((N, 8), jnp.float32),
            pltpu.VMEM((RPT, 128), jnp.int32),
            pltpu.VMEM((RPT, 128), jnp.int32),
            pltpu.VMEM((128, 16), jnp.float32),
            pltpu.VMEM((4, RPT, 128), jnp.float32),
            pltpu.VMEM((16,), jnp.float32),
            pltpu.VMEM((128, 16), jnp.float32),
            pltpu.VMEM_SHARED((NP, 16), jnp.float32),
        ],
    )
    def sc1(ab_hbm, src_hbm, dst_hbm, m_hbm, ex_hbm, den_hbm,
            ab_v, src_v, dst_v, exrow, exh, m_v, zbuf, den_sh):
        c = lax.axis_index("c")
        s = lax.axis_index("s")
        wid = c * 16 + s
        pltpu.sync_copy(ab_hbm, ab_v)
        pltpu.sync_copy(m_hbm, m_v)
        pltpu.sync_copy(src_hbm.at[pl.ds(wid * RPT, RPT)], src_v)
        pltpu.sync_copy(dst_hbm.at[pl.ds(wid * RPT, RPT)], dst_v)
        z16 = jnp.zeros((16,), jnp.float32)

        def _zb(j, carry):
            zbuf[j, :] = z16
            return carry

        lax.fori_loop(0, 128, _zb, 0)

        def _ze(j, carry):
            exrow[j, :] = z16
            return carry

        lax.fori_loop(0, 128, _ze, 0)
        for k in range(5):
            pltpu.sync_copy(zbuf, den_sh.at[pl.ds(s * STR + k * 128, 128)])
        plsc.subcore_barrier()

        iota16 = lax.iota(jnp.int32, 16)
        mreg = m_v[...]

        def _row(j, carry):
            row = wid * RPT + j
            for g in range(8):
                sidx = src_v[j, pl.ds(g * 16, 16)]
                didx = dst_v[j, pl.ds(g * 16, 16)]
                eid = row * 128 + g * 16 + iota16
                valid = eid < E
                for h in range(H):
                    va = plsc.load_gather(
                        ab_v, [sidx, jnp.full((16,), h, jnp.int32)])
                    vb = plsc.load_gather(
                        ab_v, [didx, jnp.full((16,), 4 + h, jnp.int32)])
                    al = va + vb
                    al = jnp.where(al > 0, al, al * NEG)
                    mb = jnp.full((16,), mreg[h], jnp.float32)
                    ex = jnp.exp(al - mb)
                    ex = jnp.where(valid, ex, 0.0)
                    exh[h, j, pl.ds(g * 16, 16)] = ex
                    plsc.store_scatter(
                        exrow,
                        [g * 16 + iota16, jnp.full((16,), h, jnp.int32)],
                        ex)
            pltpu.sync_copy(exrow, den_sh.at[dst_v.at[j]], add=True)
            return carry

        lax.fori_loop(0, RPT, _row, 0)
        for h in range(4):
            pltpu.sync_copy(exh.at[min(h, H - 1)],
                            ex_hbm.at[h, pl.ds(wid * RPT, RPT)])
        plsc.subcore_barrier()
        for k in range(5):
            pltpu.sync_copy(den_sh.at[pl.ds(s * STR + k * 128, 128)],
                            den_hbm.at[c, pl.ds(s * STR + k * 128, 128)])

    return sc1


_sc1_h4 = _make_sc1(4)
_sc1_h1 = _make_sc1(1)


@functools.partial(
    pl.kernel,
    out_type=jax.ShapeDtypeStruct((2, 4, NP, 128), jnp.float32),
    mesh=_MESH,
    compiler_params=pltpu.CompilerParams(needs_layout_passes=False, use_tc_tiling_on_sc=False),
    scratch_types=[
        pltpu.VMEM((RPH, 128), jnp.int32),
        pltpu.VMEM((RPH, 128), jnp.int32),
        pltpu.VMEM((128, 128), jnp.float32),
        pltpu.VMEM((128, 128), jnp.float32),
        pltpu.VMEM((RPH, 128), jnp.float32),
        pltpu.VMEM_SHARED((NP, 128), jnp.float32),
        pltpu.SemaphoreType.DMA,
        pltpu.SemaphoreType.DMA,
        pltpu.SemaphoreType.DMA,
        pltpu.SemaphoreType.DMA,
    ],
)
def _sc2(tbl_hbm, src_hbm, dst_hbm, ex_hbm, acc_hbm,
         dst_v, fidx, rows_a, rows_b, exb, acc_sh,
         sga, sgb, ssa, ssb):
    c = lax.axis_index("c")
    s = lax.axis_index("s")
    base_row = c * (ROWS // 2) + s * RPH
    HALF = RPH // 2
    z16 = jnp.zeros((16,), jnp.float32)

    def _scale(rb, j):
        def gb(g, carry):
            exv = exb[j, pl.ds(g * 16, 16)]
            for l in range(16):
                bc = jnp.full((16,), exv[l], jnp.float32)
                for sl in range(8):
                    rb[g * 16 + l, pl.ds(sl * 16, 16)] = (
                        rb[g * 16 + l, pl.ds(sl * 16, 16)] * bc)
            return carry
        lax.fori_loop(0, 8, gb, 0)

    pltpu.sync_copy(dst_hbm.at[pl.ds(base_row, RPH)], dst_v)

    def _pass(p, pcarry):
        def _zb(j, carry):
            for sl in range(8):
                rows_a[j, pl.ds(sl * 16, 16)] = z16
            return carry

        lax.fori_loop(0, 128, _zb, 0)
        for k in range(5):
            pltpu.sync_copy(rows_a, acc_sh.at[pl.ds(s * STR + k * 128, 128)])
        pltpu.sync_copy(ex_hbm.at[p, pl.ds(base_row, RPH)], exb)
        pltpu.sync_copy(src_hbm.at[pl.ds(base_row, RPH)], fidx)

        def _fb(j, carry):
            for g in range(8):
                sv = fidx[j, pl.ds(g * 16, 16)]
                fidx[j, pl.ds(g * 16, 16)] = sv * 4 + p
            return carry

        lax.fori_loop(0, RPH, _fb, 0)
        plsc.subcore_barrier()

        pltpu.async_copy(tbl_hbm.at[fidx.at[0]], rows_a, sga)

        def _it(i, carry):
            j0 = 2 * i
            j1 = 2 * i + 1
            pltpu.make_async_copy(tbl_hbm.at[fidx.at[j0]], rows_a, sga).wait()

            @pl.when(i > 0)
            def _():
                pltpu.make_async_copy(
                    rows_b, acc_sh.at[dst_v.at[j1]], ssb).wait()

            pltpu.async_copy(tbl_hbm.at[fidx.at[j1]], rows_b, sgb)
            _scale(rows_a, j0)
            pltpu.async_copy(rows_a, acc_sh.at[dst_v.at[j0]], ssa, add=True)
            pltpu.make_async_copy(tbl_hbm.at[fidx.at[j1]], rows_b, sgb).wait()
            _scale(rows_b, j1)
            pltpu.make_async_copy(rows_a, acc_sh.at[dst_v.at[j0]], ssa).wait()

            @pl.when(i < HALF - 1)
            def _():
                pltpu.async_copy(tbl_hbm.at[fidx.at[j0 + 2]], rows_a, sga)

            pltpu.async_copy(rows_b, acc_sh.at[dst_v.at[j1]], ssb, add=True)
            return carry

        lax.fori_loop(0, HALF, _it, 0)
        pltpu.make_async_copy(rows_b, acc_sh.at[dst_v.at[0]], ssb).wait()
        plsc.subcore_barrier()
        for k in range(5):
            pltpu.sync_copy(acc_sh.at[pl.ds(s * STR + k * 128, 128)],
                            acc_hbm.at[c, p, pl.ds(s * STR + k * 128, 128)])
        plsc.subcore_barrier()
        return pcarry

    lax.fori_loop(0, 4, _pass, 0)


def _mvec(m_out, H):
    ms = m_out[0, :4] + m_out[0, 4:]
    ms = jnp.where(ms > 0, ms, ms * NEG)
    return jnp.zeros((16,), jnp.float32).at[:4].set(
        jnp.where(jnp.arange(4) < H, ms, 0.0))


def kernel(x, edge_index, W1, a_src1, a_dst1, b1, W2, a_src2, a_dst2, b2):
    src = edge_index[0]
    dst = edge_index[1]
    padz = (jnp.arange(E, EP, dtype=jnp.int32) * 61) % N
    src2d = jnp.concatenate([src, padz]).reshape(ROWS, 128)
    dst2d = jnp.concatenate([dst, padz]).reshape(ROWS, 128)

    a1s = a_src1.reshape(1, 512)
    a1d = a_dst1.reshape(1, 512)
    xw1, ab1, m1 = _tc_a(x, W1, a1s, a1d)
    ex1, den1 = _sc1_h4(ab1, src2d, dst2d, _mvec(m1, 4))
    acc1 = _sc2(xw1.reshape(N * 4, 128), src2d, dst2d, ex1)

    xw2, ab2, m2 = _tc_b(acc1, den1, b1.reshape(1, 512),
                         W2.reshape(4, 128, 512),
                         a_src2.reshape(1, 512), a_dst2.reshape(1, 512))
    ex2, den2 = _sc1_h1(ab2, src2d, dst2d, _mvec(m2, 1))
    acc2 = _sc2(xw2.reshape(N * 4, 128), src2d, dst2d, ex2)

    return _tc_c(acc2, den2, b2.reshape(1, 512))
